# baseline (speedup 1.0000x reference)
import functools

import jax
import jax.numpy as jnp
from jax import lax
from jax.experimental import pallas as pl
from jax.experimental.pallas import tpu as pltpu
from jax.experimental.pallas import tpu_sc as plsc

_V = 1000000
_D = 64
_R = 4096
_C = 200
_NW = 32
_NBUF = 2

_QB = 20
_NCH = _V // 8 // _QB
_A_GROUPS = (_NCH + _NW * _NBUF - 1) // (_NW * _NBUF)

_BLK = 128


def _make_pack():
    mesh = plsc.VectorSubcoreMesh(core_axis_name="c", subcore_axis_name="s")

    @functools.partial(
        pl.kernel,
        mesh=mesh,
        out_type=jax.ShapeDtypeStruct((_V // 2, 128), jnp.float32),
        scratch_types=[
            pltpu.VMEM((_NBUF, _QB, 8, _D), jnp.float32),
            pltpu.VMEM((_NBUF, _QB * 4, 128), jnp.float32),
        ] + [pltpu.SemaphoreType.DMA] * (2 * _NBUF),
        compiler_params=pltpu.CompilerParams(
            disable_bounds_checks=True,
            disable_semaphore_checks=True,
        ),
    )
    def pack(tbl3, out, stage_a, stage_b, *sems):
        sem_i = sems[:_NBUF]
        sem_o = sems[_NBUF:]
        wid = lax.axis_index("s") * 2 + lax.axis_index("c")

        def i_copy(ch, b):
            return pltpu.make_async_copy(
                tbl3.at[pl.ds(ch * _QB, _QB)], stage_a.at[b], sem_i[b])

        def o_copy(ch, b):
            return pltpu.make_async_copy(
                stage_b.at[b],
                out.at[pl.ds(ch * _QB * 4, _QB * 4)], sem_o[b])

        def repack(b):
            def row(q, carry):
                for s in range(8):
                    for g in range(_D // 16):
                        stage_b[b, 4 * q + s // 2,
                                pl.ds((s % 2) * _D + g * 16, 16)] = (
                            stage_a[b, q, s, pl.ds(g * 16, 16)])
                return carry
            lax.fori_loop(0, _QB, row, 0)

        for b in range(_NBUF):
            ch = wid + _NW * b
            @pl.when(ch < _NCH)
            def _():
                i_copy(ch, b).start()

        def group(k, carry):
            for b in range(_NBUF):
                ch = wid + _NW * (k * _NBUF + b)
                pch = ch - _NW * _NBUF
                nch = ch + _NW * _NBUF

                @pl.when(ch < _NCH)
                def _():
                    i_copy(ch, b).wait()

                @pl.when((pch >= 0) & (pch < _NCH))
                def _():
                    o_copy(ch, b).wait()

                @pl.when(ch < _NCH)
                def _():
                    repack(b)
                    o_copy(ch, b).start()

                @pl.when(nch < _NCH)
                def _():
                    i_copy(nch, b).start()
            return carry

        lax.fori_loop(0, _A_GROUPS, group, 0)

        last_m = _A_GROUPS * _NBUF
        for m in range(last_m - _NBUF, last_m):
            ch = wid + _NW * m
            @pl.when(ch < _NCH)
            def _():
                o_copy(ch, m % _NBUF).wait()

    return pack


def _make_gather():
    mesh = plsc.VectorSubcoreMesh(core_axis_name="c", subcore_axis_name="s")

    @functools.partial(
        pl.kernel,
        mesh=mesh,
        out_type=jax.ShapeDtypeStruct((_C, _D, _R), jnp.float32),
        scratch_types=[
            pltpu.VMEM((_C, _BLK), jnp.int32),
            pltpu.VMEM((_NBUF, _BLK), jnp.int32),
            pltpu.VMEM((_NBUF, _BLK, 128), jnp.float32),
            pltpu.VMEM((_NBUF, _D, _BLK), jnp.float32),
        ] + [pltpu.SemaphoreType.DMA] * (1 + 2 * _NBUF),
        compiler_params=pltpu.CompilerParams(
            needs_layout_passes=False,
            disable_bounds_checks=True,
            disable_semaphore_checks=True,
        ),
    )
    def gather(x_t, tableR, out, idx_all, qidx, rows_w, tstage, *sems):
        sem_x = sems[0]
        sem_g = sems[1:1 + _NBUF]
        sem_o = sems[1 + _NBUF:]
        wid = lax.axis_index("s") * 2 + lax.axis_index("c")
        i0 = wid * _BLK

        pltpu.sync_copy(x_t.at[:, pl.ds(i0, _BLK)], idx_all)

        def qprep(j, b):
            for g in range(_BLK // 16):
                v = idx_all[j, pl.ds(g * 16, 16)]
                qidx[b, pl.ds(g * 16, 16)] = v >> 1

        def g_copy(b):
            return pltpu.make_async_copy(
                tableR.at[qidx.at[b]], rows_w.at[b], sem_g[b])

        def o_copy(j, b):
            return pltpu.make_async_copy(
                tstage.at[b], out.at[j, :, pl.ds(i0, _BLK)], sem_o[b])

        def transpose(j, b):
            iota = lax.iota(jnp.int32, 16)
            for g in range(_BLK // 16):
                v = idx_all[j, pl.ds(g * 16, 16)]
                rvec = iota + (g * 16)
                cbase = (v & 1) * _D

                def dloop(d, carry):
                    vals = plsc.load_gather(rows_w.at[b], [rvec, cbase + d])
                    tstage[b, d, pl.ds(g * 16, 16)] = vals
                    return carry
                lax.fori_loop(0, _D, dloop, 0)

        for b in range(_NBUF):
            qprep(b, b)
            g_copy(b).start()

        def group(k, carry):
            for b in range(_NBUF):
                j = k * _NBUF + b
                nj = j + _NBUF
                g_copy(b).wait()

                @pl.when(j >= _NBUF)
                def _():
                    o_copy(j, b).wait()

                transpose(j, b)
                o_copy(j, b).start()

                @pl.when(nj < _C)
                def _():
                    qprep(nj, b)
                    g_copy(b).start()
            return carry

        lax.fori_loop(0, _C // _NBUF, group, 0)

        for b in range(_NBUF):
            o_copy(_C - _NBUF + b, b).wait()

    return gather


_pack = _make_pack()
_gather = _make_gather()


def kernel(x, table):
    tbl3 = table.reshape(_V // 8, 8, _D)
    tableR = _pack(tbl3)
    x_t = jnp.transpose(x)
    out_t = _gather(x_t, tableR)
    return jnp.transpose(out_t, (2, 0, 1))


# speedup vs baseline: 1.0318x; 1.0318x over previous
import functools

import jax
import jax.numpy as jnp
from jax import lax
from jax.experimental import pallas as pl
from jax.experimental.pallas import tpu as pltpu
from jax.experimental.pallas import tpu_sc as plsc

_V = 1000000
_D = 64
_R = 4096
_C = 200
_NW = 32
_NBUF = 2

_QB = 20
_NCH = _V // 8 // _QB
_A_GROUPS = (_NCH + _NW * _NBUF - 1) // (_NW * _NBUF)

_BLK = 128


def _make_pack():
    mesh = plsc.VectorSubcoreMesh(core_axis_name="c", subcore_axis_name="s")

    @functools.partial(
        pl.kernel,
        mesh=mesh,
        out_type=jax.ShapeDtypeStruct((_V // 2, 128), jnp.float32),
        scratch_types=[
            pltpu.VMEM((_NBUF, _QB, 8, _D), jnp.float32),
            pltpu.VMEM((_NBUF, _QB * 4, 128), jnp.float32),
        ] + [pltpu.SemaphoreType.DMA] * (2 * _NBUF),
        compiler_params=pltpu.CompilerParams(
            disable_bounds_checks=True,
            disable_semaphore_checks=True,
        ),
    )
    def pack(tbl3, out, stage_a, stage_b, *sems):
        sem_i = sems[:_NBUF]
        sem_o = sems[_NBUF:]
        wid = lax.axis_index("s") * 2 + lax.axis_index("c")

        def i_copy(ch, b):
            return pltpu.make_async_copy(
                tbl3.at[pl.ds(ch * _QB, _QB)], stage_a.at[b], sem_i[b])

        def o_copy(ch, b):
            return pltpu.make_async_copy(
                stage_b.at[b],
                out.at[pl.ds(ch * _QB * 4, _QB * 4)], sem_o[b])

        def repack(b):
            for q in range(_QB):
                for s in range(8):
                    for g in range(_D // 16):
                        stage_b[b, 4 * q + s // 2,
                                pl.ds((s % 2) * _D + g * 16, 16)] = (
                            stage_a[b, q, s, pl.ds(g * 16, 16)])

        for b in range(_NBUF):
            ch = wid + _NW * b
            @pl.when(ch < _NCH)
            def _():
                i_copy(ch, b).start()

        def group(k, carry):
            for b in range(_NBUF):
                ch = wid + _NW * (k * _NBUF + b)
                pch = ch - _NW * _NBUF
                nch = ch + _NW * _NBUF

                @pl.when(ch < _NCH)
                def _():
                    i_copy(ch, b).wait()

                @pl.when((pch >= 0) & (pch < _NCH))
                def _():
                    o_copy(ch, b).wait()

                @pl.when(ch < _NCH)
                def _():
                    repack(b)
                    o_copy(ch, b).start()

                @pl.when(nch < _NCH)
                def _():
                    i_copy(nch, b).start()
            return carry

        lax.fori_loop(0, _A_GROUPS, group, 0)

        last_m = _A_GROUPS * _NBUF
        for m in range(last_m - _NBUF, last_m):
            ch = wid + _NW * m
            @pl.when(ch < _NCH)
            def _():
                o_copy(ch, m % _NBUF).wait()

    return pack


def _make_gather():
    mesh = plsc.VectorSubcoreMesh(core_axis_name="c", subcore_axis_name="s")

    @functools.partial(
        pl.kernel,
        mesh=mesh,
        out_type=jax.ShapeDtypeStruct((_C, _D, _R), jnp.float32),
        scratch_types=[
            pltpu.VMEM((_C, _BLK), jnp.int32),
            pltpu.VMEM((_NBUF, _BLK), jnp.int32),
            pltpu.VMEM((_NBUF, _BLK, 128), jnp.float32),
            pltpu.VMEM((_NBUF, _D, _BLK), jnp.float32),
        ] + [pltpu.SemaphoreType.DMA] * (1 + 2 * _NBUF),
        compiler_params=pltpu.CompilerParams(
            needs_layout_passes=False,
            disable_bounds_checks=True,
            disable_semaphore_checks=True,
        ),
    )
    def gather(x_t, tableR, out, idx_all, qidx, rows_w, tstage, *sems):
        sem_x = sems[0]
        sem_g = sems[1:1 + _NBUF]
        sem_o = sems[1 + _NBUF:]
        wid = lax.axis_index("s") * 2 + lax.axis_index("c")
        i0 = wid * _BLK

        pltpu.sync_copy(x_t.at[:, pl.ds(i0, _BLK)], idx_all)

        def qprep(j, b):
            for g in range(_BLK // 16):
                v = idx_all[j, pl.ds(g * 16, 16)]
                qidx[b, pl.ds(g * 16, 16)] = v >> 1

        def g_copy(b):
            return pltpu.make_async_copy(
                tableR.at[qidx.at[b]], rows_w.at[b], sem_g[b])

        def o_copy(j, b):
            return pltpu.make_async_copy(
                tstage.at[b], out.at[j, :, pl.ds(i0, _BLK)], sem_o[b])

        def transpose(j, b):
            iota = lax.iota(jnp.int32, 16)
            for g in range(_BLK // 16):
                v = idx_all[j, pl.ds(g * 16, 16)]
                rvec = iota + (g * 16)
                cbase = (v & 1) * _D

                for d in range(_D):
                    vals = plsc.load_gather(rows_w.at[b], [rvec, cbase + d])
                    tstage[b, d, pl.ds(g * 16, 16)] = vals

        for b in range(_NBUF):
            qprep(b, b)
            g_copy(b).start()

        def group(k, carry):
            for b in range(_NBUF):
                j = k * _NBUF + b
                nj = j + _NBUF
                g_copy(b).wait()

                @pl.when(j >= _NBUF)
                def _():
                    o_copy(j, b).wait()

                transpose(j, b)
                o_copy(j, b).start()

                @pl.when(nj < _C)
                def _():
                    qprep(nj, b)
                    g_copy(b).start()
            return carry

        lax.fori_loop(0, _C // _NBUF, group, 0)

        for b in range(_NBUF):
            o_copy(_C - _NBUF + b, b).wait()

    return gather


_pack = _make_pack()
_gather = _make_gather()


def kernel(x, table):
    tbl3 = table.reshape(_V // 8, 8, _D)
    tableR = _pack(tbl3)
    x_t = jnp.transpose(x)
    out_t = _gather(x_t, tableR)
    return jnp.transpose(out_t, (2, 0, 1))


# speedup vs baseline: 1.0356x; 1.0037x over previous
import functools

import jax
import jax.numpy as jnp
from jax import lax
from jax.experimental import pallas as pl
from jax.experimental.pallas import tpu as pltpu
from jax.experimental.pallas import tpu_sc as plsc

_V = 1000000
_D = 64
_R = 4096
_C = 200
_NW = 32
_NBUF = 2

_QB = 20
_NCH = _V // 8 // _QB
_A_GROUPS = (_NCH + _NW * _NBUF - 1) // (_NW * _NBUF)

_BLK = 128


def _make_pack():
    mesh = plsc.VectorSubcoreMesh(core_axis_name="c", subcore_axis_name="s")

    @functools.partial(
        pl.kernel,
        mesh=mesh,
        out_type=jax.ShapeDtypeStruct((_V // 2, 128), jnp.float32),
        scratch_types=[
            pltpu.VMEM((_NBUF, _QB, 8, _D), jnp.float32),
            pltpu.VMEM((_NBUF, _QB * 4, 128), jnp.float32),
        ] + [pltpu.SemaphoreType.DMA] * (2 * _NBUF),
        compiler_params=pltpu.CompilerParams(
            disable_bounds_checks=True,
            disable_semaphore_checks=True,
        ),
    )
    def pack(tbl3, out, stage_a, stage_b, *sems):
        sem_i = sems[:_NBUF]
        sem_o = sems[_NBUF:]
        wid = lax.axis_index("s") * 2 + lax.axis_index("c")

        def i_copy(ch, b):
            return pltpu.make_async_copy(
                tbl3.at[pl.ds(ch * _QB, _QB)], stage_a.at[b], sem_i[b])

        def o_copy(ch, b):
            return pltpu.make_async_copy(
                stage_b.at[b],
                out.at[pl.ds(ch * _QB * 4, _QB * 4)], sem_o[b])

        def repack(b):
            for q in range(_QB):
                for s in range(8):
                    for g in range(_D // 16):
                        stage_b[b, 4 * q + s // 2,
                                pl.ds((s % 2) * _D + g * 16, 16)] = (
                            stage_a[b, q, s, pl.ds(g * 16, 16)])

        for b in range(_NBUF):
            ch = wid + _NW * b
            @pl.when(ch < _NCH)
            def _():
                i_copy(ch, b).start()

        def group(k, carry):
            for b in range(_NBUF):
                ch = wid + _NW * (k * _NBUF + b)
                pch = ch - _NW * _NBUF
                nch = ch + _NW * _NBUF

                @pl.when(ch < _NCH)
                def _():
                    i_copy(ch, b).wait()

                @pl.when((pch >= 0) & (pch < _NCH))
                def _():
                    o_copy(ch, b).wait()

                @pl.when(ch < _NCH)
                def _():
                    repack(b)
                    o_copy(ch, b).start()

                @pl.when(nch < _NCH)
                def _():
                    i_copy(nch, b).start()
            return carry

        lax.fori_loop(0, _A_GROUPS, group, 0)

        last_m = _A_GROUPS * _NBUF
        for m in range(last_m - _NBUF, last_m):
            ch = wid + _NW * m
            @pl.when(ch < _NCH)
            def _():
                o_copy(ch, m % _NBUF).wait()

    return pack


def _make_gather():
    mesh = plsc.VectorSubcoreMesh(core_axis_name="c", subcore_axis_name="s")

    @functools.partial(
        pl.kernel,
        mesh=mesh,
        out_type=jax.ShapeDtypeStruct((_C, _D, _R), jnp.float32),
        scratch_types=[
            pltpu.VMEM((_C, _BLK), jnp.int32),
            pltpu.VMEM((_NBUF, _BLK), jnp.int32),
            pltpu.VMEM((_NBUF, _BLK, 128), jnp.float32),
            pltpu.VMEM((_NBUF, _D, _BLK), jnp.float32),
        ] + [pltpu.SemaphoreType.DMA] * (1 + 2 * _NBUF),
        compiler_params=pltpu.CompilerParams(
            needs_layout_passes=False,
            disable_bounds_checks=True,
            disable_semaphore_checks=True,
        ),
    )
    def gather(x_t, tableR, out, idx_all, qidx, rows_w, tstage, *sems):
        sem_x = sems[0]
        sem_g = sems[1:1 + _NBUF]
        sem_o = sems[1 + _NBUF:]
        wid = lax.axis_index("s") * 2 + lax.axis_index("c")
        i0 = wid * _BLK

        pltpu.sync_copy(x_t.at[:, pl.ds(i0, _BLK)], idx_all)

        def qprep(j, b):
            for g in range(_BLK // 16):
                v = idx_all[j, pl.ds(g * 16, 16)]
                qidx[b, pl.ds(g * 16, 16)] = v >> 1

        def g_copy(b):
            return pltpu.make_async_copy(
                tableR.at[qidx.at[b]], rows_w.at[b], sem_g[b])

        def o_copy(j, b):
            return pltpu.make_async_copy(
                tstage.at[b], out.at[j, :, pl.ds(i0, _BLK)], sem_o[b])

        def transpose(j, b):
            iota = lax.iota(jnp.int32, 16)
            rvecs, cvecs = [], []
            for g in range(_BLK // 16):
                v = idx_all[j, pl.ds(g * 16, 16)]
                rvecs.append(iota + (g * 16))
                cvecs.append((v & 1) * _D)
            for d in range(_D):
                for g in range(_BLK // 16):
                    vals = plsc.load_gather(
                        rows_w.at[b], [rvecs[g], cvecs[g] + d])
                    tstage[b, d, pl.ds(g * 16, 16)] = vals

        for b in range(_NBUF):
            qprep(b, b)
            g_copy(b).start()

        def group(k, carry):
            for b in range(_NBUF):
                j = k * _NBUF + b
                nj = j + _NBUF
                g_copy(b).wait()

                @pl.when(j >= _NBUF)
                def _():
                    o_copy(j, b).wait()

                transpose(j, b)
                o_copy(j, b).start()

                @pl.when(nj < _C)
                def _():
                    qprep(nj, b)
                    g_copy(b).start()
            return carry

        lax.fori_loop(0, _C // _NBUF, group, 0)

        for b in range(_NBUF):
            o_copy(_C - _NBUF + b, b).wait()

    return gather


_pack = _make_pack()
_gather = _make_gather()


def kernel(x, table):
    tbl3 = table.reshape(_V // 8, 8, _D)
    tableR = _pack(tbl3)
    x_t = jnp.transpose(x)
    out_t = _gather(x_t, tableR)
    return jnp.transpose(out_t, (2, 0, 1))


# speedup vs baseline: 1.7921x; 1.7305x over previous
import functools

import jax
import jax.numpy as jnp
from jax import lax
from jax.experimental import pallas as pl
from jax.experimental.pallas import tpu as pltpu
from jax.experimental.pallas import tpu_sc as plsc

_V = 1000000
_D = 64
_R = 4096
_C = 200
_NW = 32

_ABUF = 2
_QB = 20
_NCH = _V // 8 // _QB
_A_GROUPS = (_NCH + _NW * _ABUF - 1) // (_NW * _ABUF)

_BLK = 128
_GBUF = 4


def _make_widen():
    mesh = plsc.VectorSubcoreMesh(core_axis_name="c", subcore_axis_name="s")

    @functools.partial(
        pl.kernel,
        mesh=mesh,
        out_type=jax.ShapeDtypeStruct((_V // 8, 8, 128), jnp.float32),
        scratch_types=[
            pltpu.VMEM((_ABUF, _QB, 8, _D), jnp.float32),
            pltpu.VMEM((_ABUF, _QB, 8, 128), jnp.float32),
        ] + [pltpu.SemaphoreType.DMA] * (2 * _ABUF),
        compiler_params=pltpu.CompilerParams(
            disable_bounds_checks=True,
            disable_semaphore_checks=True,
        ),
    )
    def widen(tbl3, out, stage_a, stage_b, *sems):
        sem_i = sems[:_ABUF]
        sem_o = sems[_ABUF:]
        wid = lax.axis_index("s") * 2 + lax.axis_index("c")

        def i_copy(ch, b):
            return pltpu.make_async_copy(
                tbl3.at[pl.ds(ch * _QB, _QB)], stage_a.at[b], sem_i[b])

        def o_copy(ch, b):
            return pltpu.make_async_copy(
                stage_b.at[b], out.at[pl.ds(ch * _QB, _QB)], sem_o[b])

        def repack(b):
            for q in range(_QB):
                for s in range(8):
                    for g in range(_D // 16):
                        stage_b[b, q, s, pl.ds(g * 16, 16)] = (
                            stage_a[b, q, s, pl.ds(g * 16, 16)])

        for b in range(_ABUF):
            ch = wid + _NW * b
            @pl.when(ch < _NCH)
            def _():
                i_copy(ch, b).start()

        def group(k, carry):
            for b in range(_ABUF):
                ch = wid + _NW * (k * _ABUF + b)
                pch = ch - _NW * _ABUF
                nch = ch + _NW * _ABUF

                @pl.when(ch < _NCH)
                def _():
                    i_copy(ch, b).wait()

                @pl.when((pch >= 0) & (pch < _NCH))
                def _():
                    o_copy(ch, b).wait()

                @pl.when(ch < _NCH)
                def _():
                    repack(b)
                    o_copy(ch, b).start()

                @pl.when(nch < _NCH)
                def _():
                    i_copy(nch, b).start()
            return carry

        lax.fori_loop(0, _A_GROUPS, group, 0)

        last_m = _A_GROUPS * _ABUF
        for m in range(last_m - _ABUF, last_m):
            ch = wid + _NW * m
            @pl.when(ch < _NCH)
            def _():
                o_copy(ch, m % _ABUF).wait()

    return widen


def _make_gather():
    mesh = plsc.VectorSubcoreMesh(core_axis_name="c", subcore_axis_name="s")

    @functools.partial(
        pl.kernel,
        mesh=mesh,
        out_type=jax.ShapeDtypeStruct((_R, _C, 128), jnp.float32),
        scratch_types=[
            pltpu.VMEM((_C, _BLK), jnp.int32),
            pltpu.VMEM((_GBUF, _BLK, 128), jnp.float32),
        ] + [pltpu.SemaphoreType.DMA] * (2 * _GBUF),
        compiler_params=pltpu.CompilerParams(
            disable_bounds_checks=True,
            disable_semaphore_checks=True,
        ),
    )
    def gather(x_t, tableP, out, idx_all, rows_w, *sems):
        sem_g = sems[:_GBUF]
        sem_o = sems[_GBUF:]
        wid = lax.axis_index("s") * 2 + lax.axis_index("c")
        i0 = wid * _BLK

        pltpu.sync_copy(x_t.at[:, pl.ds(i0, _BLK)], idx_all)

        def g_copy(j, b):
            return pltpu.make_async_copy(
                tableP.at[idx_all.at[j]], rows_w.at[b], sem_g[b])

        def o_copy(j, b):
            return pltpu.make_async_copy(
                rows_w.at[b], out.at[pl.ds(i0, _BLK), j], sem_o[b])

        for b in range(_GBUF):
            g_copy(b, b).start()

        def group(k, carry):
            j0 = k * _GBUF
            for b in range(_GBUF):
                g_copy(j0 + b, b).wait()
                o_copy(j0 + b, b).start()
            for b in range(_GBUF):
                o_copy(j0 + b, b).wait()

                @pl.when(j0 + _GBUF + b < _C)
                def _():
                    g_copy(j0 + _GBUF + b, b).start()
            return carry

        lax.fori_loop(0, _C // _GBUF, group, 0)

    return gather


_widen = _make_widen()
_gather = _make_gather()


def kernel(x, table):
    tbl3 = table.reshape(_V // 8, 8, _D)
    tableP = _widen(tbl3).reshape(_V, 128)
    x_t = jnp.transpose(x)
    out_wide = _gather(x_t, tableP)
    return out_wide[:, :, :_D]


# speedup vs baseline: 1.7964x; 1.0024x over previous
import functools

import jax
import jax.numpy as jnp
from jax import lax
from jax.experimental import pallas as pl
from jax.experimental.pallas import tpu as pltpu
from jax.experimental.pallas import tpu_sc as plsc

_V = 1000000
_D = 64
_R = 4096
_C = 200
_NW = 32

_ABUF = 2
_QB = 25
_NCH = _V // 8 // _QB
_A_GROUPS = (_NCH + _NW * _ABUF - 1) // (_NW * _ABUF)

_BLK = 128
_GBUF = 5


def _make_widen():
    mesh = plsc.VectorSubcoreMesh(core_axis_name="c", subcore_axis_name="s")

    @functools.partial(
        pl.kernel,
        mesh=mesh,
        out_type=jax.ShapeDtypeStruct((_V // 8, 8, 128), jnp.float32),
        scratch_types=[
            pltpu.VMEM((_ABUF, _QB, 8, _D), jnp.float32),
            pltpu.VMEM((_ABUF, _QB, 8, 128), jnp.float32),
        ] + [pltpu.SemaphoreType.DMA] * (2 * _ABUF),
        compiler_params=pltpu.CompilerParams(
            disable_bounds_checks=True,
            disable_semaphore_checks=True,
        ),
    )
    def widen(tbl3, out, stage_a, stage_b, *sems):
        sem_i = sems[:_ABUF]
        sem_o = sems[_ABUF:]
        wid = lax.axis_index("s") * 2 + lax.axis_index("c")

        def i_copy(ch, b):
            return pltpu.make_async_copy(
                tbl3.at[pl.ds(ch * _QB, _QB)], stage_a.at[b], sem_i[b])

        def o_copy(ch, b):
            return pltpu.make_async_copy(
                stage_b.at[b], out.at[pl.ds(ch * _QB, _QB)], sem_o[b])

        def repack(b):
            for q in range(_QB):
                for s in range(8):
                    for g in range(_D // 16):
                        stage_b[b, q, s, pl.ds(g * 16, 16)] = (
                            stage_a[b, q, s, pl.ds(g * 16, 16)])

        for b in range(_ABUF):
            ch = wid + _NW * b
            @pl.when(ch < _NCH)
            def _():
                i_copy(ch, b).start()

        def group(k, carry):
            for b in range(_ABUF):
                ch = wid + _NW * (k * _ABUF + b)
                pch = ch - _NW * _ABUF
                nch = ch + _NW * _ABUF

                @pl.when(ch < _NCH)
                def _():
                    i_copy(ch, b).wait()

                @pl.when((pch >= 0) & (pch < _NCH))
                def _():
                    o_copy(ch, b).wait()

                @pl.when(ch < _NCH)
                def _():
                    repack(b)
                    o_copy(ch, b).start()

                @pl.when(nch < _NCH)
                def _():
                    i_copy(nch, b).start()
            return carry

        lax.fori_loop(0, _A_GROUPS, group, 0)

        last_m = _A_GROUPS * _ABUF
        for m in range(last_m - _ABUF, last_m):
            ch = wid + _NW * m
            @pl.when(ch < _NCH)
            def _():
                o_copy(ch, m % _ABUF).wait()

    return widen


def _make_gather():
    mesh = plsc.VectorSubcoreMesh(core_axis_name="c", subcore_axis_name="s")

    @functools.partial(
        pl.kernel,
        mesh=mesh,
        out_type=jax.ShapeDtypeStruct((_R, _C, 128), jnp.float32),
        scratch_types=[
            pltpu.VMEM((_C, _BLK), jnp.int32),
            pltpu.VMEM((_GBUF, _BLK, 128), jnp.float32),
        ] + [pltpu.SemaphoreType.DMA] * (2 * _GBUF),
        compiler_params=pltpu.CompilerParams(
            disable_bounds_checks=True,
            disable_semaphore_checks=True,
        ),
    )
    def gather(x_t, tableP, out, idx_all, rows_w, *sems):
        sem_g = sems[:_GBUF]
        sem_o = sems[_GBUF:]
        wid = lax.axis_index("s") * 2 + lax.axis_index("c")
        i0 = wid * _BLK

        pltpu.sync_copy(x_t.at[:, pl.ds(i0, _BLK)], idx_all)

        def g_copy(j, b):
            return pltpu.make_async_copy(
                tableP.at[idx_all.at[j]], rows_w.at[b], sem_g[b])

        def o_copy(j, b):
            return pltpu.make_async_copy(
                rows_w.at[b], out.at[pl.ds(i0, _BLK), j], sem_o[b])

        for b in range(_GBUF):
            g_copy(b, b).start()

        def group(k, carry):
            j0 = k * _GBUF
            for b in range(_GBUF):
                g_copy(j0 + b, b).wait()
                o_copy(j0 + b, b).start()
            for b in range(_GBUF):
                o_copy(j0 + b, b).wait()

                @pl.when(j0 + _GBUF + b < _C)
                def _():
                    g_copy(j0 + _GBUF + b, b).start()
            return carry

        lax.fori_loop(0, _C // _GBUF, group, 0)

    return gather


_widen = _make_widen()
_gather = _make_gather()


def kernel(x, table):
    tbl3 = table.reshape(_V // 8, 8, _D)
    tableP = _widen(tbl3).reshape(_V, 128)
    x_t = jnp.transpose(x)
    out_wide = _gather(x_t, tableP)
    return out_wide[:, :, :_D]


# speedup vs baseline: 1.7970x; 1.0003x over previous
import functools

import jax
import jax.numpy as jnp
from jax import lax
from jax.experimental import pallas as pl
from jax.experimental.pallas import tpu as pltpu
from jax.experimental.pallas import tpu_sc as plsc

_V = 1000000
_D = 64
_R = 4096
_C = 200
_NW = 32

_ABUF = 3
_QB = 20
_NCH = _V // 8 // _QB
_A_GROUPS = (_NCH + _NW * _ABUF - 1) // (_NW * _ABUF)

_BLK = 128
_GBUF = 5


def _make_widen():
    mesh = plsc.VectorSubcoreMesh(core_axis_name="c", subcore_axis_name="s")

    @functools.partial(
        pl.kernel,
        mesh=mesh,
        out_type=jax.ShapeDtypeStruct((_V // 8, 8, 128), jnp.float32),
        scratch_types=[
            pltpu.VMEM((_ABUF, _QB, 8, _D), jnp.float32),
            pltpu.VMEM((_ABUF, _QB, 8, 128), jnp.float32),
        ] + [pltpu.SemaphoreType.DMA] * (2 * _ABUF),
        compiler_params=pltpu.CompilerParams(
            disable_bounds_checks=True,
            disable_semaphore_checks=True,
        ),
    )
    def widen(tbl3, out, stage_a, stage_b, *sems):
        sem_i = sems[:_ABUF]
        sem_o = sems[_ABUF:]
        wid = lax.axis_index("s") * 2 + lax.axis_index("c")

        def i_copy(ch, b):
            return pltpu.make_async_copy(
                tbl3.at[pl.ds(ch * _QB, _QB)], stage_a.at[b], sem_i[b])

        def o_copy(ch, b):
            return pltpu.make_async_copy(
                stage_b.at[b], out.at[pl.ds(ch * _QB, _QB)], sem_o[b])

        def repack(b):
            for q in range(_QB):
                for s in range(8):
                    for g in range(_D // 16):
                        stage_b[b, q, s, pl.ds(g * 16, 16)] = (
                            stage_a[b, q, s, pl.ds(g * 16, 16)])

        for b in range(_ABUF):
            ch = wid + _NW * b
            @pl.when(ch < _NCH)
            def _():
                i_copy(ch, b).start()

        def group(k, carry):
            for b in range(_ABUF):
                ch = wid + _NW * (k * _ABUF + b)
                pch = ch - _NW * _ABUF
                nch = ch + _NW * _ABUF

                @pl.when(ch < _NCH)
                def _():
                    i_copy(ch, b).wait()

                @pl.when((pch >= 0) & (pch < _NCH))
                def _():
                    o_copy(ch, b).wait()

                @pl.when(ch < _NCH)
                def _():
                    repack(b)
                    o_copy(ch, b).start()

                @pl.when(nch < _NCH)
                def _():
                    i_copy(nch, b).start()
            return carry

        lax.fori_loop(0, _A_GROUPS, group, 0)

        last_m = _A_GROUPS * _ABUF
        for m in range(last_m - _ABUF, last_m):
            ch = wid + _NW * m
            @pl.when(ch < _NCH)
            def _():
                o_copy(ch, m % _ABUF).wait()

    return widen


def _make_gather():
    mesh = plsc.VectorSubcoreMesh(core_axis_name="c", subcore_axis_name="s")

    @functools.partial(
        pl.kernel,
        mesh=mesh,
        out_type=jax.ShapeDtypeStruct((_R, _C, 128), jnp.float32),
        scratch_types=[
            pltpu.VMEM((_C, _BLK), jnp.int32),
            pltpu.VMEM((_GBUF, _BLK, 128), jnp.float32),
        ] + [pltpu.SemaphoreType.DMA] * (2 * _GBUF),
        compiler_params=pltpu.CompilerParams(
            disable_bounds_checks=True,
            disable_semaphore_checks=True,
        ),
    )
    def gather(x_t, tableP, out, idx_all, rows_w, *sems):
        sem_g = sems[:_GBUF]
        sem_o = sems[_GBUF:]
        wid = lax.axis_index("s") * 2 + lax.axis_index("c")
        i0 = wid * _BLK

        pltpu.sync_copy(x_t.at[:, pl.ds(i0, _BLK)], idx_all)

        def g_copy(j, b):
            return pltpu.make_async_copy(
                tableP.at[idx_all.at[j]], rows_w.at[b], sem_g[b])

        def o_copy(j, b):
            return pltpu.make_async_copy(
                rows_w.at[b], out.at[pl.ds(i0, _BLK), j], sem_o[b])

        for b in range(_GBUF):
            g_copy(b, b).start()

        def group(k, carry):
            j0 = k * _GBUF
            for b in range(_GBUF):
                g_copy(j0 + b, b).wait()
                o_copy(j0 + b, b).start()
            for b in range(_GBUF):
                o_copy(j0 + b, b).wait()

                @pl.when(j0 + _GBUF + b < _C)
                def _():
                    g_copy(j0 + _GBUF + b, b).start()
            return carry

        lax.fori_loop(0, _C // _GBUF, group, 0)

    return gather


_widen = _make_widen()
_gather = _make_gather()


def kernel(x, table):
    tbl3 = table.reshape(_V // 8, 8, _D)
    tableP = _widen(tbl3).reshape(_V, 128)
    x_t = jnp.transpose(x)
    out_wide = _gather(x_t, tableP)
    return out_wide[:, :, :_D]
